# initial kernel scaffold (unmeasured)
import jax
import jax.numpy as jnp
from jax import lax
from jax.experimental import pallas as pl
from jax.experimental.pallas import tpu as pltpu

N_DEV = 4
B = 2
S = 512
HQ = 8
DH = 64
HD = HQ * DH
E = 768
WINDOW = 128
NGLOB = 32
SCALE = 0.125


def _allgather_kv(k2, v2):

    def body(k_ref, v_ref, kout_ref, vout_ref, comm_ref, send_sems, recv_sems):
        my = lax.axis_index("i")
        left = lax.rem(my + N_DEV - 1, N_DEV)
        right = lax.rem(my + 1, N_DEV)

        barrier_sem = pltpu.get_barrier_semaphore()
        for nbr in (left, right):
            pl.semaphore_signal(
                barrier_sem, inc=1,
                device_id=(nbr,), device_id_type=pl.DeviceIdType.MESH,
            )
        pl.semaphore_wait(barrier_sem, 2)

        k_bf = k_ref[...].astype(jnp.bfloat16)
        v_bf = v_ref[...].astype(jnp.bfloat16)
        comm_ref[0, :, :, 0:HD] = k_bf
        comm_ref[0, :, :, HD:] = v_bf
        kout_ref[:, pl.ds(my * S, S), :] = k_bf
        vout_ref[:, pl.ds(my * S, S), :] = v_bf

        for h in range(N_DEV - 1):
            rdma = pltpu.make_async_remote_copy(
                src_ref=comm_ref.at[h],
                dst_ref=comm_ref.at[h + 1],
                send_sem=send_sems.at[h],
                recv_sem=recv_sems.at[h],
                device_id=(right,),
                device_id_type=pl.DeviceIdType.MESH,
            )
            rdma.start()
            rdma.wait()
            origin = lax.rem(my + N_DEV - 1 - h, N_DEV)
            kout_ref[:, pl.ds(origin * S, S), :] = comm_ref[h + 1, :, :, 0:HD]
            vout_ref[:, pl.ds(origin * S, S), :] = comm_ref[h + 1, :, :, HD:]

    out_shape = jax.ShapeDtypeStruct((B, N_DEV * S, HD), jnp.bfloat16)
    return pl.pallas_call(
        body,
        out_shape=(out_shape, out_shape),
        in_specs=[
            pl.BlockSpec(memory_space=pltpu.VMEM),
            pl.BlockSpec(memory_space=pltpu.VMEM),
        ],
        out_specs=(
            pl.BlockSpec(memory_space=pltpu.VMEM),
            pl.BlockSpec(memory_space=pltpu.VMEM),
        ),
        scratch_shapes=[
            pltpu.VMEM((N_DEV, B, S, 2 * HD), jnp.bfloat16),
            pltpu.SemaphoreType.DMA((N_DEV - 1,)),
            pltpu.SemaphoreType.DMA((N_DEV - 1,)),
        ],
        compiler_params=pltpu.CompilerParams(collective_id=0),
    )(k2, v2)


def _attention(x, Wq, k_full, v_full, Wo):
    skv = N_DEV * S

    def body(x_ref, wq_ref, k_ref, v_ref, wo_ref, out_ref):
        h = pl.program_id(1)
        my = lax.axis_index("i")

        xb = x_ref[0].astype(jnp.bfloat16)
        wq = wq_ref[...].astype(jnp.bfloat16)
        q = jnp.dot(xb, wq, preferred_element_type=jnp.float32)

        k = k_ref[0]
        s = lax.dot_general(
            q.astype(jnp.bfloat16), k,
            (((1,), (1,)), ((), ())),
            preferred_element_type=jnp.float32,
        ) * SCALE

        qi = my * S + lax.broadcasted_iota(jnp.int32, (S, skv), 0)
        ki = lax.broadcasted_iota(jnp.int32, (S, skv), 1)
        mask = (jnp.abs(qi - ki) <= WINDOW) | (ki < NGLOB) | (qi < NGLOB)
        s = jnp.where(mask, s, -1e9)

        m = jnp.max(s, axis=1, keepdims=True)
        w = jnp.exp(s - m)
        w = w / jnp.sum(w, axis=1, keepdims=True)

        ctx = jnp.dot(w.astype(jnp.bfloat16), v_ref[0],
                      preferred_element_type=jnp.float32)
        o = jnp.dot(ctx.astype(jnp.bfloat16),
                    wo_ref[...].astype(jnp.bfloat16),
                    preferred_element_type=jnp.float32)

        @pl.when(h == 0)
        def _():
            out_ref[0] = o

        @pl.when(h > 0)
        def _():
            out_ref[0] += o

    return pl.pallas_call(
        body,
        grid=(B, HQ),
        in_specs=[
            pl.BlockSpec((1, S, E), lambda b, h: (b, 0, 0)),
            pl.BlockSpec((E, DH), lambda b, h: (0, h)),
            pl.BlockSpec((1, skv, DH), lambda b, h: (b, 0, h)),
            pl.BlockSpec((1, skv, DH), lambda b, h: (b, 0, h)),
            pl.BlockSpec((DH, E), lambda b, h: (h, 0)),
        ],
        out_specs=pl.BlockSpec((1, S, E), lambda b, h: (b, 0, 0)),
        out_shape=jax.ShapeDtypeStruct((B, S, E), jnp.float32),
    )(x, Wq, k_full, v_full, Wo)


def kernel(x, Wq, K_ext, V_ext, Wo):
    k2 = K_ext.reshape(B, S, HD)
    v2 = V_ext.reshape(B, S, HD)
    k_full, v_full = _allgather_kv(k2, v2)
    return _attention(x, Wq, k_full, v_full, Wo)


# baseline (device time: 126827 ns/iter reference)
import jax
import jax.numpy as jnp
from jax import lax
from jax.experimental import pallas as pl
from jax.experimental.pallas import tpu as pltpu

N_DEV = 4
B = 2
S = 512
HQ = 8
DH = 64
HD = HQ * DH
E = 768
WINDOW = 128
NGLOB = 32
SCALE = 0.125


def _allgather_kv(k2, v2):

    def body(k_ref, v_ref, kout_ref, vout_ref, comm_ref, send_sems, recv_sems):
        my = lax.axis_index("i")
        left = lax.rem(my + N_DEV - 1, N_DEV)
        right = lax.rem(my + 1, N_DEV)

        barrier_sem = pltpu.get_barrier_semaphore()
        for nbr in (left, right):
            pl.semaphore_signal(
                barrier_sem, inc=1,
                device_id=(nbr,), device_id_type=pl.DeviceIdType.MESH,
            )
        pl.semaphore_wait(barrier_sem, 2)

        k_bf = k_ref[...].astype(jnp.bfloat16)
        v_bf = v_ref[...].astype(jnp.bfloat16)
        comm_ref[0, :, :, 0:HD] = k_bf
        comm_ref[0, :, :, HD:] = v_bf
        kout_ref[:, pl.ds(my * S, S), :] = k_bf
        vout_ref[:, pl.ds(my * S, S), :] = v_bf

        for h in range(N_DEV - 1):
            rdma = pltpu.make_async_remote_copy(
                src_ref=comm_ref.at[h],
                dst_ref=comm_ref.at[h + 1],
                send_sem=send_sems.at[h],
                recv_sem=recv_sems.at[h],
                device_id=(right,),
                device_id_type=pl.DeviceIdType.MESH,
            )
            rdma.start()
            rdma.wait()
            origin = lax.rem(my + N_DEV - 1 - h, N_DEV)
            kout_ref[:, pl.ds(origin * S, S), :] = comm_ref[h + 1, :, :, 0:HD]
            vout_ref[:, pl.ds(origin * S, S), :] = comm_ref[h + 1, :, :, HD:]

    out_shape = jax.ShapeDtypeStruct((B, N_DEV * S, HD), jnp.bfloat16)
    return pl.pallas_call(
        body,
        out_shape=(out_shape, out_shape),
        in_specs=[
            pl.BlockSpec(memory_space=pltpu.VMEM),
            pl.BlockSpec(memory_space=pltpu.VMEM),
        ],
        out_specs=(
            pl.BlockSpec(memory_space=pltpu.VMEM),
            pl.BlockSpec(memory_space=pltpu.VMEM),
        ),
        scratch_shapes=[
            pltpu.VMEM((N_DEV, B, S, 2 * HD), jnp.bfloat16),
            pltpu.SemaphoreType.DMA((N_DEV - 1,)),
            pltpu.SemaphoreType.DMA((N_DEV - 1,)),
        ],
        compiler_params=pltpu.CompilerParams(collective_id=0),
    )(k2, v2)


def _attention(x, Wq, k_full, v_full, Wo):
    skv = N_DEV * S

    def body(x_ref, wq_ref, k_ref, v_ref, wo_ref, out_ref):
        my = lax.axis_index("i")

        xb = x_ref[0].astype(jnp.bfloat16)
        wq = wq_ref[...].astype(jnp.bfloat16)
        wo = wo_ref[...].astype(jnp.bfloat16)
        k = k_ref[0]
        v = v_ref[0]

        qi = my * S + lax.broadcasted_iota(jnp.int32, (S, skv), 0)
        ki = lax.broadcasted_iota(jnp.int32, (S, skv), 1)
        mask = (jnp.abs(qi - ki) <= WINDOW) | (ki < NGLOB) | (qi < NGLOB)

        acc = jnp.zeros((S, E), jnp.float32)
        for h in range(HQ):
            sl = slice(h * DH, (h + 1) * DH)
            q = jnp.dot(xb, wq[:, sl],
                        preferred_element_type=jnp.float32)
            s = lax.dot_general(
                q.astype(jnp.bfloat16), k[:, sl],
                (((1,), (1,)), ((), ())),
                preferred_element_type=jnp.float32,
            ) * SCALE
            s = jnp.where(mask, s, -1e9)
            m = jnp.max(s, axis=1, keepdims=True)
            w = jnp.exp(s - m)
            w = w / jnp.sum(w, axis=1, keepdims=True)
            ctx = jnp.dot(w.astype(jnp.bfloat16), v[:, sl],
                          preferred_element_type=jnp.float32)
            acc += jnp.dot(ctx.astype(jnp.bfloat16), wo[sl, :],
                           preferred_element_type=jnp.float32)
        out_ref[0] = acc

    return pl.pallas_call(
        body,
        grid=(B,),
        in_specs=[
            pl.BlockSpec((1, S, E), lambda b: (b, 0, 0)),
            pl.BlockSpec((E, HD), lambda b: (0, 0)),
            pl.BlockSpec((1, skv, HD), lambda b: (b, 0, 0)),
            pl.BlockSpec((1, skv, HD), lambda b: (b, 0, 0)),
            pl.BlockSpec((HD, E), lambda b: (0, 0)),
        ],
        out_specs=pl.BlockSpec((1, S, E), lambda b: (b, 0, 0)),
        out_shape=jax.ShapeDtypeStruct((B, S, E), jnp.float32),
    )(x, Wq, k_full, v_full, Wo)


def kernel(x, Wq, K_ext, V_ext, Wo):
    k2 = K_ext.reshape(B, S, HD)
    v2 = V_ext.reshape(B, S, HD)
    k_full, v_full = _allgather_kv(k2, v2)
    return _attention(x, Wq, k_full, v_full, Wo)


# device time: 66983 ns/iter; 1.8934x vs baseline; 1.8934x over previous
import jax
import jax.numpy as jnp
from jax import lax
from jax.experimental import pallas as pl
from jax.experimental.pallas import tpu as pltpu

N_DEV = 4
B = 2
S = 512
HQ = 8
DH = 64
HD = HQ * DH
E = 768
WINDOW = 128
NGLOB = 32
SCALE = 0.125

_RECTS_OWN = [(0, S, 0, S)]
_RECTS_LEFT = [(32, 96, S - WINDOW, WINDOW), (0, S, 0, NGLOB),
               (0, NGLOB, NGLOB, S - NGLOB)]
_RECTS_RIGHT = [(S - WINDOW, WINDOW, NGLOB, 96), (0, S, 0, NGLOB),
                (0, NGLOB, NGLOB, S - NGLOB)]
_RECTS_FAR = [(0, S, 0, NGLOB), (0, NGLOB, NGLOB, S - NGLOB)]


def _fused(x, Wq, k2, v2, Wo):

    def body(x_ref, wq_ref, k2_ref, v2_ref, wo_ref, out_ref,
             kbuf, vbuf, q_ref, acc_ref, l_ref, send_sems, recv_sems):
        my = lax.axis_index("i")
        left = lax.rem(my + N_DEV - 1, N_DEV)
        right = lax.rem(my + 1, N_DEV)

        barrier_sem = pltpu.get_barrier_semaphore()
        for nbr in (left, right):
            pl.semaphore_signal(
                barrier_sem, inc=1,
                device_id=(nbr,), device_id_type=pl.DeviceIdType.MESH,
            )

        kbuf[0] = k2_ref[...].astype(jnp.bfloat16)
        vbuf[0] = v2_ref[...].astype(jnp.bfloat16)
        pl.semaphore_wait(barrier_sem, 2)

        def rdma(src, dst, i, dev):
            return pltpu.make_async_remote_copy(
                src_ref=src, dst_ref=dst,
                send_sem=send_sems.at[i], recv_sem=recv_sems.at[i],
                device_id=(dev,), device_id_type=pl.DeviceIdType.MESH,
            )

        p1kr = rdma(kbuf.at[0], kbuf.at[1], 0, right)
        p1vr = rdma(vbuf.at[0], vbuf.at[1], 1, right)
        p1kl = rdma(kbuf.at[0], kbuf.at[2], 2, left)
        p1vl = rdma(vbuf.at[0], vbuf.at[2], 3, left)
        p2k = rdma(kbuf.at[1], kbuf.at[3], 4, right)
        p2v = rdma(vbuf.at[2], vbuf.at[3], 5, left)

        p1kr.start()
        p1vr.start()
        p1kl.start()
        p1vl.start()

        wq = wq_ref[...].astype(jnp.bfloat16)
        for b in range(B):
            q_ref[b] = jnp.dot(
                x_ref[b].astype(jnp.bfloat16), wq,
                preferred_element_type=jnp.float32,
            ).astype(jnp.bfloat16)

        def process(slot, cbase, rects, init):
            for b in range(B):
                for r0, nr, c0, nc in rects:
                    qi = my * S + r0 + lax.broadcasted_iota(
                        jnp.int32, (nr, nc), 0)
                    ki = cbase + c0 + lax.broadcasted_iota(
                        jnp.int32, (nr, nc), 1)
                    mask = ((jnp.abs(qi - ki) <= WINDOW)
                            | (ki < NGLOB) | (qi < NGLOB))
                    kb = kbuf[slot, b, c0:c0 + nc, :]
                    vb = vbuf[slot, b, c0:c0 + nc, :]
                    for h in range(HQ):
                        sl = slice(h * DH, (h + 1) * DH)
                        q = q_ref[b, r0:r0 + nr, sl]
                        s = lax.dot_general(
                            q, kb[:, sl],
                            (((1,), (1,)), ((), ())),
                            preferred_element_type=jnp.float32,
                        ) * SCALE
                        w = jnp.exp(jnp.where(mask, s, -1e9))
                        lsum = jnp.sum(w, axis=1, keepdims=True)
                        pv = jnp.dot(w.astype(jnp.bfloat16), vb[:, sl],
                                     preferred_element_type=jnp.float32)
                        if init:
                            l_ref[b, r0:r0 + nr, h:h + 1] = lsum
                            acc_ref[b, r0:r0 + nr, sl] = pv
                        else:
                            l_ref[b, r0:r0 + nr, h:h + 1] += lsum
                            acc_ref[b, r0:r0 + nr, sl] += pv

        process(0, my * S, _RECTS_OWN, init=True)

        p1kr.wait_recv()
        p2k.start()
        p1vr.wait_recv()
        process(1, left * S, _RECTS_LEFT, init=False)

        p1kl.wait_recv()
        p1vl.wait_recv()
        p2v.start()
        process(2, right * S, _RECTS_RIGHT, init=False)

        p2k.wait_recv()
        p2v.wait_recv()
        process(3, lax.rem(my + 2, N_DEV) * S, _RECTS_FAR, init=False)

        wo = wo_ref[...].astype(jnp.bfloat16)
        for b in range(B):
            lb = l_ref[b]
            o = jnp.zeros((S, E), jnp.float32)
            for h in range(HQ):
                sl = slice(h * DH, (h + 1) * DH)
                ctx = acc_ref[b, :, sl] / lb[:, h:h + 1]
                o += jnp.dot(ctx.astype(jnp.bfloat16), wo[sl, :],
                             preferred_element_type=jnp.float32)
            out_ref[b] = o

        p1kr.wait_send()
        p1vr.wait_send()
        p1kl.wait_send()
        p1vl.wait_send()
        p2k.wait_send()
        p2v.wait_send()

    return pl.pallas_call(
        body,
        out_shape=jax.ShapeDtypeStruct((B, S, E), jnp.float32),
        in_specs=[pl.BlockSpec(memory_space=pltpu.VMEM)] * 5,
        out_specs=pl.BlockSpec(memory_space=pltpu.VMEM),
        scratch_shapes=[
            pltpu.VMEM((N_DEV, B, S, HD), jnp.bfloat16),
            pltpu.VMEM((N_DEV, B, S, HD), jnp.bfloat16),
            pltpu.VMEM((B, S, HD), jnp.bfloat16),
            pltpu.VMEM((B, S, HD), jnp.float32),
            pltpu.VMEM((B, S, HQ), jnp.float32),
            pltpu.SemaphoreType.DMA((6,)),
            pltpu.SemaphoreType.DMA((6,)),
        ],
        compiler_params=pltpu.CompilerParams(collective_id=0),
    )(x, Wq, k2, v2, Wo)


def kernel(x, Wq, K_ext, V_ext, Wo):
    k2 = K_ext.reshape(B, S, HD)
    v2 = V_ext.reshape(B, S, HD)
    return _fused(x, Wq, k2, v2, Wo)


# device time: 40183 ns/iter; 3.1562x vs baseline; 1.6669x over previous
import functools
import os

import jax
import jax.numpy as jnp
from jax import lax
from jax.experimental import pallas as pl
from jax.experimental.pallas import tpu as pltpu

_MODE = os.environ.get("SPARSE_KERNEL_MODE", "")
_DO_COMM = _MODE != "compute_only"
_DO_COMPUTE = _MODE != "comm_only"

N_DEV = 4
B = 2
S = 512
HQ = 8
DH = 64
HD = HQ * DH
E = 768
WINDOW = 128
NGLOB = 32
SCALE = 0.125


def _fused(x, Wq, k2, v2, Wo):

    def body(x_ref, wq_ref, k2_ref, v2_ref, wo_ref, out_ref,
             kown, vown, q_ref, acc_ref, l_ref,
             hlk, hlv, hrk, hrv, gk, gv, gq,
             pacc_s, pl_s, pacc_r, pl_r,
             send_sems, recv_sems):
        my = lax.axis_index("i")
        left = lax.rem(my + N_DEV - 1, N_DEV)
        right = lax.rem(my + 1, N_DEV)

        barrier_sem = pltpu.get_barrier_semaphore()
        for d in range(N_DEV):
            pl.semaphore_signal(
                barrier_sem, inc=1,
                device_id=(jnp.int32(d),),
                device_id_type=pl.DeviceIdType.MESH,
            )
        pl.semaphore_wait(barrier_sem, N_DEV)

        for b in range(B):
            kown[b] = k2_ref[b].astype(jnp.bfloat16)
            vown[b] = v2_ref[b].astype(jnp.bfloat16)

        def rdma(src, dst, si, ri, dev):
            return pltpu.make_async_remote_copy(
                src_ref=src, dst_ref=dst,
                send_sem=send_sems.at[si], recv_sem=recv_sems.at[ri],
                device_id=(dev,), device_id_type=pl.DeviceIdType.MESH,
            )

        hkr = rdma(kown.at[:, pl.ds(S - WINDOW, WINDOW), :], hlk, 0, 0, right)
        hvr = rdma(vown.at[:, pl.ds(S - WINDOW, WINDOW), :], hlv, 1, 1, right)
        hkl = rdma(kown.at[:, pl.ds(0, WINDOW), :], hrk, 2, 2, left)
        hvl = rdma(vown.at[:, pl.ds(0, WINDOW), :], hrv, 3, 3, left)
        if _DO_COMM:
            hkr.start()
            hvr.start()
            hkl.start()
            hvl.start()

            @pl.when(my == 0)
            def _():
                for j, t in enumerate((1, 2, 3)):
                    rdma(kown.at[:, pl.ds(0, NGLOB), :], gk,
                         4 + 3 * j, 4, t).start()
                    rdma(vown.at[:, pl.ds(0, NGLOB), :], gv,
                         5 + 3 * j, 5, t).start()

        wq = wq_ref[...].astype(jnp.bfloat16)
        for b in range(B):
            q_ref[b] = jnp.dot(
                x_ref[b].astype(jnp.bfloat16), wq,
                preferred_element_type=jnp.float32,
            ).astype(jnp.bfloat16)

        if _DO_COMM:
            @pl.when(my == 0)
            def _():
                for j, t in enumerate((1, 2, 3)):
                    rdma(q_ref.at[:, pl.ds(0, NGLOB), :], gq,
                         6 + 3 * j, 6, t).start()

        def process(kref, vref, ncols, cbase, rects, mask_mode, init):
            del ncols
            for b in range(B):
                for r0, nr, c0, nc in rects:
                    qi = my * S + r0 + lax.broadcasted_iota(
                        jnp.int32, (nr, nc), 0)
                    ki = cbase + c0 + lax.broadcasted_iota(
                        jnp.int32, (nr, nc), 1)
                    if mask_mode == "full":
                        mask = ((jnp.abs(qi - ki) <= WINDOW)
                                | (ki < NGLOB) | (qi < NGLOB))
                    elif mask_mode == "window32":
                        mask = (jnp.abs(qi - ki) <= WINDOW) & (qi >= NGLOB)
                    else:
                        mask = None
                    kb = kref[b, c0:c0 + nc, :]
                    vb = vref[b, c0:c0 + nc, :]
                    for h in range(HQ):
                        sl = slice(h * DH, (h + 1) * DH)
                        s = lax.dot_general(
                            q_ref[b, r0:r0 + nr, sl], kb[:, sl],
                            (((1,), (1,)), ((), ())),
                            preferred_element_type=jnp.float32,
                        ) * SCALE
                        if mask is not None:
                            s = jnp.where(mask, s, -1e9)
                        w = jnp.exp(s)
                        lsum = jnp.sum(w, axis=1, keepdims=True)
                        pv = jnp.dot(w.astype(jnp.bfloat16), vb[:, sl],
                                     preferred_element_type=jnp.float32)
                        if init:
                            l_ref[b, r0:r0 + nr, h:h + 1] = lsum
                            acc_ref[b, r0:r0 + nr, sl] = pv
                        else:
                            l_ref[b, r0:r0 + nr, h:h + 1] += lsum
                            acc_ref[b, r0:r0 + nr, sl] += pv

        if _DO_COMPUTE:
            process(kown, vown, S, my * S, [(0, S, 0, S)], "full", True)

        grecv_k = rdma(kown.at[:, pl.ds(0, NGLOB), :], gk, 4, 4, 0)
        grecv_v = rdma(vown.at[:, pl.ds(0, NGLOB), :], gv, 4, 5, 0)
        grecv_q = rdma(q_ref.at[:, pl.ds(0, NGLOB), :], gq, 4, 6, 0)
        psend_a = rdma(pacc_s, pacc_r.at[my - 1], 4, 5 + 2 * my, 0)
        psend_l = rdma(pl_s, pl_r.at[my - 1], 5, 6 + 2 * my, 0)

        @pl.when(my != 0)
        def _():
            if _DO_COMM:
                grecv_q.wait_recv()
            if _DO_COMPUTE:
                for b in range(B):
                    for h in range(HQ):
                        sl = slice(h * DH, (h + 1) * DH)
                        s = lax.dot_general(
                            gq[b, :, sl], kown[b, :, sl],
                            (((1,), (1,)), ((), ())),
                            preferred_element_type=jnp.float32,
                        ) * SCALE
                        w = jnp.exp(s)
                        pl_s[b, :, h:h + 1] = jnp.sum(w, 1, keepdims=True)
                        pacc_s[b, :, sl] = jnp.dot(
                            w.astype(jnp.bfloat16), vown[b, :, sl],
                            preferred_element_type=jnp.float32)
            if _DO_COMM:
                psend_a.start()
                psend_l.start()

        if _DO_COMM:
            hkr.wait_recv()
            hvr.wait_recv()
            hkl.wait_recv()
            hvl.wait_recv()
        if _DO_COMPUTE:
            process(hlk, hlv, WINDOW, left * S + (S - WINDOW),
                    [(0, WINDOW, 0, WINDOW)], "window32", False)
            process(hrk, hrv, WINDOW, right * S,
                    [(S - WINDOW, WINDOW, 0, WINDOW)], "window32", False)

        @pl.when(my != 0)
        def _():
            if _DO_COMM:
                grecv_k.wait_recv()
                grecv_v.wait_recv()
            if _DO_COMPUTE:
                process(gk, gv, NGLOB, 0, [(0, S, 0, NGLOB)], "none", False)

        @pl.when(my == 0)
        def _():
            if _DO_COMM:
                for j in range(3):
                    rdma(pacc_s, pacc_r.at[j], 4, 7 + 2 * j, 0).wait_recv()
                    rdma(pl_s, pl_r.at[j], 5, 8 + 2 * j, 0).wait_recv()
            if _DO_COMPUTE:
                for j in range(3):
                    for b in range(B):
                        acc_ref[b, 0:NGLOB, :] += pacc_r[j, b]
                        l_ref[b, 0:NGLOB, :] += pl_r[j, b]

        wo = wo_ref[...].astype(jnp.bfloat16)
        for b in range(B):
            lb = l_ref[b]
            o = jnp.zeros((S, E), jnp.float32)
            for h in range(HQ):
                sl = slice(h * DH, (h + 1) * DH)
                ctx = acc_ref[b, :, sl] / lb[:, h:h + 1]
                o += jnp.dot(ctx.astype(jnp.bfloat16),
                             wo[sl, :],
                             preferred_element_type=jnp.float32)
            out_ref[b] = o

        if _DO_COMM:
            hkr.wait_send()
            hvr.wait_send()
            hkl.wait_send()
            hvl.wait_send()

            @pl.when(my == 0)
            def _():
                for j in range(3):
                    rdma(kown.at[:, pl.ds(0, NGLOB), :], gk,
                         4 + 3 * j, 4, 1).wait_send()
                    rdma(vown.at[:, pl.ds(0, NGLOB), :], gv,
                         5 + 3 * j, 5, 1).wait_send()
                    rdma(q_ref.at[:, pl.ds(0, NGLOB), :], gq,
                         6 + 3 * j, 6, 1).wait_send()

            @pl.when(my != 0)
            def _():
                psend_a.wait_send()
                psend_l.wait_send()

        @functools.partial(pl.run_scoped,
                           second_barrier=pltpu.SemaphoreType.REGULAR)
        def _(second_barrier):
            for d in range(N_DEV):
                pl.semaphore_signal(
                    second_barrier, inc=1,
                    device_id=(jnp.int32(d),),
                    device_id_type=pl.DeviceIdType.MESH,
                )
            pl.semaphore_wait(second_barrier, N_DEV)

    return pl.pallas_call(
        body,
        out_shape=jax.ShapeDtypeStruct((B, S, E), jnp.float32),
        in_specs=[pl.BlockSpec(memory_space=pltpu.VMEM)] * 5,
        out_specs=pl.BlockSpec(memory_space=pltpu.VMEM),
        scratch_shapes=[
            pltpu.VMEM((B, S, HD), jnp.bfloat16),
            pltpu.VMEM((B, S, HD), jnp.bfloat16),
            pltpu.VMEM((B, S, HD), jnp.bfloat16),
            pltpu.VMEM((B, S, HD), jnp.float32),
            pltpu.VMEM((B, S, HQ), jnp.float32),
            pltpu.VMEM((B, WINDOW, HD), jnp.bfloat16),
            pltpu.VMEM((B, WINDOW, HD), jnp.bfloat16),
            pltpu.VMEM((B, WINDOW, HD), jnp.bfloat16),
            pltpu.VMEM((B, WINDOW, HD), jnp.bfloat16),
            pltpu.VMEM((B, NGLOB, HD), jnp.bfloat16),
            pltpu.VMEM((B, NGLOB, HD), jnp.bfloat16),
            pltpu.VMEM((B, NGLOB, HD), jnp.bfloat16),
            pltpu.VMEM((B, NGLOB, HD), jnp.float32),
            pltpu.VMEM((B, NGLOB, HQ), jnp.float32),
            pltpu.VMEM((3, B, NGLOB, HD), jnp.float32),
            pltpu.VMEM((3, B, NGLOB, HQ), jnp.float32),
            pltpu.SemaphoreType.DMA((13,)),
            pltpu.SemaphoreType.DMA((13,)),
        ],
        compiler_params=pltpu.CompilerParams(collective_id=0),
    )(x, Wq, k2, v2, Wo)


def kernel(x, Wq, K_ext, V_ext, Wo):
    k2 = K_ext.reshape(B, S, HD)
    v2 = V_ext.reshape(B, S, HD)
    return _fused(x, Wq, k2, v2, Wo)
